# baseline (speedup 1.0000x reference)
import functools

import jax
import jax.numpy as jnp
from jax.experimental import pallas as pl
from jax.experimental.pallas import tpu as pltpu
from jax.experimental.pallas import tpu_sc as plsc

NUM_CORES = 2
NUM_SUBCORES = 16
CHUNK = 400
BLOCK_M = 2048


def _gather_sc(table2, idx_half):
    n = idx_half.shape[0]
    width = table2.shape[1]
    nw = NUM_CORES * NUM_SUBCORES
    b_per_w = n // nw
    n_chunks = b_per_w // CHUNK

    mesh = plsc.VectorSubcoreMesh(core_axis_name="c", subcore_axis_name="s")

    @functools.partial(
        pl.kernel, mesh=mesh,
        out_type=jax.ShapeDtypeStruct((n, width), table2.dtype),
        scratch_types=[
            pltpu.VMEM((CHUNK,), jnp.int32),
            pltpu.VMEM((CHUNK, width), jnp.float32),
            pltpu.SemaphoreType.DMA,
        ],
    )
    def gather_kernel(table_hbm, idx_hbm, out_hbm, idx_v, rows_v, sem):
        wid = jax.lax.axis_index("s") * NUM_CORES + jax.lax.axis_index("c")
        base_w = wid * b_per_w

        @pl.loop(0, n_chunks)
        def _(g):
            base = base_w + g * CHUNK
            pltpu.sync_copy(idx_hbm.at[pl.ds(base, CHUNK)], idx_v)
            pltpu.async_copy(table_hbm.at[idx_v], rows_v, sem).wait()
            pltpu.sync_copy(rows_v, out_hbm.at[pl.ds(base, CHUNK)])

    return gather_kernel(table2, idx_half)


def _select_matmul_tc(pairs, parity, W, b):
    M = pairs.shape[0]
    K, N = W.shape

    def mm_kernel(pairs_ref, par_ref, w_ref, b_ref, out_ref):
        p = par_ref[...]
        emb = pairs_ref[:, :K] * (1.0 - p) + pairs_ref[:, K:] * p
        out_ref[...] = jnp.dot(emb, w_ref[...],
                               preferred_element_type=jnp.float32) + b_ref[...]

    return pl.pallas_call(
        mm_kernel,
        out_shape=jax.ShapeDtypeStruct((M, N), jnp.float32),
        grid=(M // BLOCK_M,),
        in_specs=[
            pl.BlockSpec((BLOCK_M, 2 * K), lambda i: (i, 0)),
            pl.BlockSpec((BLOCK_M, 1), lambda i: (i, 0)),
            pl.BlockSpec((K, N), lambda i: (0, 0)),
            pl.BlockSpec((1, N), lambda i: (0, 0)),
        ],
        out_specs=pl.BlockSpec((BLOCK_M, N), lambda i: (i, 0)),
    )(pairs, parity, W, b.reshape(1, N))


def kernel(token_ids, table, W, b):
    B, L = token_ids.shape
    flat_ids = token_ids.reshape(-1)
    idx_half = jax.lax.shift_right_logical(flat_ids, 1)
    parity = (flat_ids & 1).astype(jnp.float32).reshape(-1, 1)
    table2 = table.reshape(table.shape[0] // 2, 2 * table.shape[1])
    pairs = _gather_sc(table2, idx_half)
    out = _select_matmul_tc(pairs, parity, W, b)
    return out.reshape(B, L, W.shape[1])


# speedup vs baseline: 1.0393x; 1.0393x over previous
import functools

import jax
import jax.numpy as jnp
from jax.experimental import pallas as pl
from jax.experimental.pallas import tpu as pltpu
from jax.experimental.pallas import tpu_sc as plsc

NUM_CORES = 2
NUM_SUBCORES = 16
CHUNK = 800
BLOCK_M = 2048


def _gather_sc(table, flat_ids):
    n = flat_ids.shape[0]
    width = table.shape[1]
    nw = NUM_CORES * NUM_SUBCORES
    b_per_w = n // nw
    n_chunks = b_per_w // CHUNK

    mesh = plsc.VectorSubcoreMesh(core_axis_name="c", subcore_axis_name="s")

    @functools.partial(
        pl.kernel, mesh=mesh,
        out_type=jax.ShapeDtypeStruct((n, width), table.dtype),
        scratch_types=[
            pltpu.VMEM((CHUNK,), jnp.int32),
            pltpu.VMEM((CHUNK, width), jnp.float32),
            pltpu.SemaphoreType.DMA,
        ],
        compiler_params=pltpu.CompilerParams(use_tc_tiling_on_sc=False),
    )
    def gather_kernel(table_hbm, idx_hbm, out_hbm, idx_v, rows_v, sem):
        wid = jax.lax.axis_index("s") * NUM_CORES + jax.lax.axis_index("c")
        base_w = wid * b_per_w

        @pl.loop(0, n_chunks)
        def _(g):
            base = base_w + g * CHUNK
            pltpu.sync_copy(idx_hbm.at[pl.ds(base, CHUNK)], idx_v)
            pltpu.async_copy(table_hbm.at[idx_v], rows_v, sem).wait()
            pltpu.sync_copy(rows_v, out_hbm.at[pl.ds(base, CHUNK)])

    return gather_kernel(table, flat_ids)


def _matmul_tc(emb, W, b):
    M, K = emb.shape
    N = W.shape[1]

    def mm_kernel(emb_ref, w_ref, b_ref, out_ref):
        out_ref[...] = jnp.dot(emb_ref[...], w_ref[...],
                               preferred_element_type=jnp.float32) + b_ref[...]

    return pl.pallas_call(
        mm_kernel,
        out_shape=jax.ShapeDtypeStruct((M, N), jnp.float32),
        grid=(M // BLOCK_M,),
        in_specs=[
            pl.BlockSpec((BLOCK_M, K), lambda i: (i, 0)),
            pl.BlockSpec((K, N), lambda i: (0, 0)),
            pl.BlockSpec((1, N), lambda i: (0, 0)),
        ],
        out_specs=pl.BlockSpec((BLOCK_M, N), lambda i: (i, 0)),
    )(emb, W, b.reshape(1, N))


def kernel(token_ids, table, W, b):
    B, L = token_ids.shape
    flat_ids = token_ids.reshape(-1)
    emb = _gather_sc(table, flat_ids)
    out = _matmul_tc(emb, W, b)
    return out.reshape(B, L, W.shape[1])


# speedup vs baseline: 2.5250x; 2.4296x over previous
import functools

import jax
import jax.numpy as jnp
from jax.experimental import pallas as pl
from jax.experimental.pallas import tpu as pltpu
from jax.experimental.pallas import tpu_sc as plsc

NUM_CORES = 2
NUM_SUBCORES = 16
CHUNK = 800
BLOCK_M = 2048
REPACK_BK = 8064
HALF = REPACK_BK // 2


def _repack_tc(tableT):
    K, V = tableT.shape
    n_blocks = pl.cdiv(V, REPACK_BK)

    def rp_kernel(x_ref, out_ref):
        y = x_ref[...].T
        out_ref[:, :K] = y[:HALF]
        out_ref[:, K:] = y[HALF:]

    return pl.pallas_call(
        rp_kernel,
        out_shape=jax.ShapeDtypeStruct((n_blocks * HALF, 2 * K), jnp.float32),
        grid=(n_blocks,),
        in_specs=[pl.BlockSpec((K, REPACK_BK), lambda i: (0, i))],
        out_specs=pl.BlockSpec((HALF, 2 * K), lambda i: (i, 0)),
    )(tableT)


def _gather_sc(table_lin, flat_ids, width_pad):
    n = flat_ids.shape[0]
    width = table_lin.shape[1]
    nw = NUM_CORES * NUM_SUBCORES
    b_per_w = n // nw
    n_chunks = b_per_w // CHUNK

    mesh = plsc.VectorSubcoreMesh(core_axis_name="c", subcore_axis_name="s")

    @functools.partial(
        pl.kernel, mesh=mesh,
        out_type=jax.ShapeDtypeStruct((n, width_pad), jnp.float32),
        scratch_types=[
            pltpu.VMEM((CHUNK,), jnp.int32),
            pltpu.VMEM((CHUNK, width), jnp.float32),
            pltpu.SemaphoreType.DMA,
        ],
        compiler_params=pltpu.CompilerParams(use_tc_tiling_on_sc=False),
    )
    def gather_kernel(table_hbm, idx_hbm, out_hbm, idx_v, rows_v, sem):
        wid = jax.lax.axis_index("s") * NUM_CORES + jax.lax.axis_index("c")
        base_w = wid * b_per_w

        @pl.loop(0, n_chunks)
        def _(g):
            base = base_w + g * CHUNK
            pltpu.sync_copy(idx_hbm.at[pl.ds(base, CHUNK)], idx_v)
            pltpu.async_copy(table_hbm.at[idx_v], rows_v, sem).wait()
            pltpu.sync_copy(rows_v,
                            out_hbm.at[pl.ds(base, CHUNK), pl.ds(0, width)])

    return gather_kernel(table_lin, flat_ids)


def _matmul_tc(emb_pad, W, b):
    M = emb_pad.shape[0]
    K, N = W.shape

    def mm_kernel(emb_ref, w_ref, b_ref, out_ref):
        x = emb_ref[:, :K].astype(jnp.bfloat16)
        w = w_ref[...].astype(jnp.bfloat16)
        out_ref[...] = jnp.dot(x, w,
                               preferred_element_type=jnp.float32) + b_ref[...]

    return pl.pallas_call(
        mm_kernel,
        out_shape=jax.ShapeDtypeStruct((M, N), jnp.float32),
        grid=(M // BLOCK_M,),
        in_specs=[
            pl.BlockSpec((BLOCK_M, emb_pad.shape[1]), lambda i: (i, 0)),
            pl.BlockSpec((K, N), lambda i: (0, 0)),
            pl.BlockSpec((1, N), lambda i: (0, 0)),
        ],
        out_specs=pl.BlockSpec((BLOCK_M, N), lambda i: (i, 0)),
    )(emb_pad, W, b.reshape(1, N))


def kernel(token_ids, table, W, b):
    B, L = token_ids.shape
    N = W.shape[1]
    idx_lmajor = token_ids.T.reshape(-1)
    p = idx_lmajor % REPACK_BK
    idx_perm = (idx_lmajor - p) + 2 * (p % HALF) + p // HALF
    pairs = _repack_tc(table.T)
    table_lin = pairs.reshape(pairs.shape[0] * 2, table.shape[1])
    emb_pad = _gather_sc(table_lin, idx_perm, 2 * table.shape[1])
    out_lmajor = _matmul_tc(emb_pad, W, b)
    return out_lmajor.reshape(L, B, N).transpose(1, 0, 2)
